# baseline (speedup 1.0000x reference)
import functools

import jax
import jax.numpy as jnp
import numpy as np
from jax.experimental import pallas as pl

EMB = 256
NUM_LAYERS = 4
NP_ = 8000
NL_ = 2000
GP_SIZES = [2000, 2000, 2000, 2000]
GL_SIZES = [500, 500, 500, 500]

BE = 2000
BN = 1000


def _silu(x):
    return x * jax.nn.sigmoid(x)


def _ab_body(h_ref, w_ref, o_ref):
    o_ref[...] = jnp.dot(h_ref[...], w_ref[...],
                         preferred_element_type=jnp.float32)


def _node_ab(h, w_ab):
    n = h.shape[0]
    return pl.pallas_call(
        _ab_body,
        grid=(n // BN,),
        in_specs=[
            pl.BlockSpec((BN, EMB), lambda i: (i, 0)),
            pl.BlockSpec((EMB, 2 * EMB), lambda i: (0, 0)),
        ],
        out_specs=pl.BlockSpec((BN, 2 * EMB), lambda i: (i, 0)),
        out_shape=jax.ShapeDtypeStruct((n, 2 * EMB), jnp.float32),
    )(h, w_ab)


def _edge_body(d_ref, s_ref, wf_ref, bf_ref, w2_ref, b2_ref, o_ref):
    f = s_ref[...] + jnp.dot(d_ref[...], wf_ref[...],
                             preferred_element_type=jnp.float32) + bf_ref[...]
    m = _silu(f)
    g = jnp.dot(m, w2_ref[...], preferred_element_type=jnp.float32) + b2_ref[...]
    o_ref[...] = _silu(g)


def _edge_mlp(d16, s, wf, bf, w2, b2):
    e = s.shape[0]
    return pl.pallas_call(
        _edge_body,
        grid=(e // BE,),
        in_specs=[
            pl.BlockSpec((BE, 16), lambda i: (i, 0)),
            pl.BlockSpec((BE, EMB), lambda i: (i, 0)),
            pl.BlockSpec((16, EMB), lambda i: (0, 0)),
            pl.BlockSpec((1, EMB), lambda i: (0, 0)),
            pl.BlockSpec((EMB, EMB), lambda i: (0, 0)),
            pl.BlockSpec((1, EMB), lambda i: (0, 0)),
        ],
        out_specs=pl.BlockSpec((BE, EMB), lambda i: (i, 0)),
        out_shape=jax.ShapeDtypeStruct((e, EMB), jnp.float32),
    )(d16, s, wf, bf, w2, b2)


def _nodeup_body(h_ref, hn_ref, w1a_ref, w1b_ref, b1_ref, w2_ref, b2_ref, o_ref):
    t = (jnp.dot(h_ref[...], w1a_ref[...], preferred_element_type=jnp.float32)
         + jnp.dot(hn_ref[...], w1b_ref[...], preferred_element_type=jnp.float32)
         + b1_ref[...])
    m = _silu(t)
    o_ref[...] = jnp.dot(m, w2_ref[...],
                         preferred_element_type=jnp.float32) + b2_ref[...]


def _node_update(h, hn, w1a, w1b, b1, w2, b2):
    n = h.shape[0]
    return pl.pallas_call(
        _nodeup_body,
        grid=(n // BN,),
        in_specs=[
            pl.BlockSpec((BN, EMB), lambda i: (i, 0)),
            pl.BlockSpec((BN, EMB), lambda i: (i, 0)),
            pl.BlockSpec((EMB, EMB), lambda i: (0, 0)),
            pl.BlockSpec((EMB, EMB), lambda i: (0, 0)),
            pl.BlockSpec((1, EMB), lambda i: (0, 0)),
            pl.BlockSpec((EMB, EMB), lambda i: (0, 0)),
            pl.BlockSpec((1, EMB), lambda i: (0, 0)),
        ],
        out_specs=pl.BlockSpec((BN, EMB), lambda i: (i, 0)),
        out_shape=jax.ShapeDtypeStruct((n, EMB), jnp.float32),
    )(h, hn, w1a, w1b, b1, w2, b2)


def _layer_norm(p, x):
    mu = jnp.mean(x, axis=-1, keepdims=True)
    var = jnp.var(x, axis=-1, keepdims=True)
    return (x - mu) / jnp.sqrt(var + 1e-5) * p['g'] + p['b']


def _fold_edge_weights(pe, p_layer):
    w1 = p_layer['e1']['w']
    b1 = p_layer['e1']['b']
    w1a = w1[:EMB]
    w1b = w1[EMB:2 * EMB]
    wr = w1[2 * EMB:2 * EMB + 1]
    w1e = w1[2 * EMB + 1:]
    k = pe['w'].shape[0]
    wf = jnp.zeros((16, EMB), jnp.float32)
    wf = wf.at[:k].set(pe['w'] @ w1e)
    wf = wf.at[15:16].set(wr)
    bf = (pe['b'] @ w1e + b1)[None, :]
    return wf, bf, jnp.concatenate([w1a, w1b], axis=1)


def _conv(h, d16, src, dst, n, pe, p):
    wf, bf, w1ab = _fold_edge_weights(pe, p)
    ab = _node_ab(h, w1ab)
    s = ab[src, :EMB] + ab[dst, EMB:]
    msg = _edge_mlp(d16, s, wf, bf, p['e2']['w'], p['e2']['b'][None, :])
    hn = jax.ops.segment_sum(msg, dst, num_segments=n)
    wn1 = p['n1']['w']
    return _node_update(h, hn, wn1[:EMB], wn1[EMB:], p['n1']['b'][None, :],
                        p['n2']['w'], p['n2']['b'][None, :])


def _d16(dist, radial):
    e = dist.shape[0]
    d = jnp.zeros((e, 16), jnp.float32)
    d = d.at[:, :dist.shape[1]].set(dist)
    return d.at[:, 15].set(radial)


def _radial(x, src, dst):
    diff = x[src] - x[dst]
    return jnp.sum(diff * diff, axis=1)


def kernel(token_res, token_atom, gp_dist, gp_pos_enc, gp_coord, gp_src, gp_dst,
           gl_feat, gl_efeat, gl_pos_enc, gl_coord, gl_src, gl_dst,
           gc_dist, gc_src, gc_dst, params):
    nc = NP_ + NL_
    hp = jnp.concatenate([params['res_emb'][token_res],
                          params['atom_emb'][token_atom]], axis=1)
    hp = _layer_norm(params['p_norm'], hp)
    hl = gl_feat @ params['l_node']['w'] + params['l_node']['b']
    hl = _layer_norm(params['l_norm'], hl)

    rp = _radial(gp_coord, gp_src, gp_dst)
    rl = _radial(gl_coord, gl_src, gl_dst)
    xc = jnp.concatenate([gp_coord, gl_coord], axis=0)
    rc = _radial(xc, gc_src, gc_dst)
    dp = _d16(gp_dist, rp)
    dl = _d16(gl_efeat, rl)
    dc = _d16(gc_dist, rc)

    def remap(c):
        k = c // 2500
        r = c - k * 2500
        return jnp.where(r < 2000, k * 2000 + r, NP_ + k * 500 + (r - 2000))
    csrc = remap(gc_src)
    cdst = remap(gc_dst)

    for li in range(NUM_LAYERS):
        hp = _conv(hp, dp, gp_src, gp_dst, NP_, params['p_edge'],
                   params['protein_block'][li])
        hl = _conv(hl, dl, gl_src, gl_dst, NL_, params['l_edge'],
                   params['ligand_block'][li])
        hc = jnp.concatenate([hp, hl], axis=0)
        hc = _conv(hc, dc, csrc, cdst, nc, params['c_edge'],
                   params['complex_block'][li])
        hp = hc[:NP_]
        hl = hc[NP_:]

    seg = np.concatenate([np.repeat(np.arange(4), 2000),
                          np.repeat(np.arange(4), 500)])
    h = jax.ops.segment_sum(hc, jnp.asarray(seg), num_segments=4)
    z = h @ params['ba1']['w'] + params['ba1']['b']
    bn = params['bn']
    z = (z - bn['rm']) / jnp.sqrt(bn['rv'] + 1e-5) * bn['g'] + bn['b']
    z = jax.nn.elu(z)
    return z @ params['ba2']['w'] + params['ba2']['b']
